# Initial kernel scaffold; baseline (speedup 1.0000x reference)
import functools

import jax
import jax.numpy as jnp
from jax import lax
from jax.experimental import pallas as pl
from jax.experimental.pallas import tpu as pltpu
from jax.experimental.pallas import tpu_sc as plsc

NC = 2
NS = 16
NW = NC * NS
CHUNK = 128


def _sc_segment_sum(h2, src_idx, dst_idx, zeros, n_pad, k, d):
  rpz = n_pad // NS
  mesh = plsc.VectorSubcoreMesh(core_axis_name="c", subcore_axis_name="s")

  @functools.partial(
      pl.kernel,
      out_type=jax.ShapeDtypeStruct((NC, n_pad, d), jnp.float32),
      mesh=mesh,
      scratch_types=[
          pltpu.VMEM((k, CHUNK), jnp.int32),
          pltpu.VMEM((k, CHUNK), jnp.int32),
          pltpu.VMEM((CHUNK, d), jnp.float32),
          pltpu.VMEM_SHARED((n_pad, d), jnp.float32),
      ],
  )
  def body(h2_hbm, src_hbm, dst_hbm, zero_hbm, out_hbm, src_v, dst_v, rows_v,
           agg_sh):
    c = lax.axis_index("c")
    s = lax.axis_index("s")
    wid = c * NS + s
    pltpu.sync_copy(src_hbm.at[wid], src_v)
    pltpu.sync_copy(dst_hbm.at[wid], dst_v)
    pltpu.sync_copy(zero_hbm, agg_sh.at[pl.ds(s * rpz, rpz)])
    plsc.subcore_barrier()

    @pl.loop(0, k)
    def _(j):
      pltpu.sync_copy(h2_hbm.at[src_v.at[j]], rows_v)
      pltpu.sync_copy(rows_v, agg_sh.at[dst_v.at[j]], add=True)

    plsc.subcore_barrier()
    pltpu.sync_copy(agg_sh.at[pl.ds(s * rpz, rpz)],
                    out_hbm.at[c, pl.ds(s * rpz, rpz)])

  return body(h2, src_idx, dst_idx, zeros)


def _sc_degree(dst_idx, ones, zeros, n_pad, k):
  rpz = n_pad // NS
  mesh = plsc.VectorSubcoreMesh(core_axis_name="c", subcore_axis_name="s")

  @functools.partial(
      pl.kernel,
      out_type=jax.ShapeDtypeStruct((NC, n_pad, 16), jnp.float32),
      mesh=mesh,
      scratch_types=[
          pltpu.VMEM((k, CHUNK), jnp.int32),
          pltpu.VMEM((CHUNK, 16), jnp.float32),
          pltpu.VMEM_SHARED((n_pad, 16), jnp.float32),
      ],
  )
  def body(dst_hbm, ones_hbm, zero_hbm, out_hbm, dst_v, ones_v, deg_sh):
    c = lax.axis_index("c")
    s = lax.axis_index("s")
    wid = c * NS + s
    pltpu.sync_copy(dst_hbm.at[wid], dst_v)
    pltpu.sync_copy(ones_hbm, ones_v)
    pltpu.sync_copy(zero_hbm, deg_sh.at[pl.ds(s * rpz, rpz)])
    plsc.subcore_barrier()

    @pl.loop(0, k)
    def _(j):
      pltpu.sync_copy(ones_v, deg_sh.at[dst_v.at[j]], add=True)

    plsc.subcore_barrier()
    pltpu.sync_copy(deg_sh.at[pl.ds(s * rpz, rpz)],
                    out_hbm.at[c, pl.ds(s * rpz, rpz)])

  return body(dst_idx, ones, zeros)


def _tc_matmul(x, w):
  def body(x_ref, w_ref, o_ref):
    o_ref[...] = jnp.dot(x_ref[...], w_ref[...],
                         preferred_element_type=jnp.float32)

  return pl.pallas_call(
      body,
      out_shape=jax.ShapeDtypeStruct((x.shape[0], w.shape[1]), jnp.float32),
  )(x, w)


def _tc_scale(p, deg, n):
  def body(p_ref, deg_ref, o_ref):
    dinv = lax.rsqrt(deg_ref[0, :n, 0:1] + deg_ref[1, :n, 0:1])
    o_ref[...] = dinv * p_ref[...]

  return pl.pallas_call(
      body,
      out_shape=jax.ShapeDtypeStruct(p.shape, jnp.float32),
  )(p, deg)


def _tc_mid(agg, deg, b, w, n):
  def body(agg_ref, deg_ref, b_ref, w_ref, o_ref):
    dinv = lax.rsqrt(deg_ref[0, :n, 0:1] + deg_ref[1, :n, 0:1])
    h = jnp.maximum(dinv * (agg_ref[0, :n, :] + agg_ref[1, :n, :])
                    + b_ref[...], 0.0)
    o_ref[...] = dinv * jnp.dot(h, w_ref[...],
                                preferred_element_type=jnp.float32)

  return pl.pallas_call(
      body,
      out_shape=jax.ShapeDtypeStruct((n, w.shape[1]), jnp.float32),
  )(agg, deg, b, w)


def _tc_final(agg, deg, b, n, d_out):
  def body(agg_ref, deg_ref, b_ref, o_ref):
    dinv = lax.rsqrt(deg_ref[0, :n, 0:1] + deg_ref[1, :n, 0:1])
    o_ref[...] = jax.nn.sigmoid(
        dinv * (agg_ref[0, :n, :d_out] + agg_ref[1, :n, :d_out]) + b_ref[...])

  return pl.pallas_call(
      body,
      out_shape=jax.ShapeDtypeStruct((n, d_out), jnp.float32),
  )(agg, deg, b)


def kernel(x, edge_index, W1, b1, W2, b2, W3, b3, W4, b4):
  n, d_in = x.shape
  e = edge_index.shape[1]
  d_hid = W1.shape[1]
  d_out = W4.shape[1]

  tot = e + n
  per = NW * CHUNK
  k = -(-tot // per)
  k = -(-k // 8) * 8
  e_pad = k * per
  n_pad = -(-n // (NS * CHUNK)) * (NS * CHUNK)

  loop = jnp.arange(n, dtype=jnp.int32)
  src = jnp.concatenate([edge_index[0].astype(jnp.int32), loop])
  dst = jnp.concatenate([edge_index[1].astype(jnp.int32), loop])
  npad_e = e_pad - tot
  pad_dst = n + (jnp.arange(npad_e, dtype=jnp.int32) % (n_pad - n))
  src = jnp.pad(src, (0, npad_e)).reshape(NW, k, CHUNK)
  dst = jnp.concatenate([dst, pad_dst]).reshape(NW, k, CHUNK)

  rpz = n_pad // NS
  zeros_hid = jnp.zeros((rpz, d_hid), jnp.float32)
  zeros_32 = jnp.zeros((rpz, 32), jnp.float32)
  zeros_16 = jnp.zeros((rpz, 16), jnp.float32)
  ones_16 = jnp.ones((CHUNK, 16), jnp.float32)

  b1r = b1.reshape(1, -1)
  b2r = b2.reshape(1, -1)
  b3r = b3.reshape(1, -1)
  b4r = b4.reshape(1, -1)
  w4p = jnp.pad(W4, ((0, 0), (0, 32 - d_out)))

  deg = _sc_degree(dst, ones_16, zeros_16, n_pad, k)
  p1 = _tc_matmul(x, W1)
  h2 = _tc_scale(p1, deg, n)
  agg = _sc_segment_sum(h2, src, dst, zeros_hid, n_pad, k, d_hid)
  h2 = _tc_mid(agg, deg, b1r, W2, n)
  agg = _sc_segment_sum(h2, src, dst, zeros_hid, n_pad, k, d_hid)
  h2 = _tc_mid(agg, deg, b2r, W3, n)
  agg = _sc_segment_sum(h2, src, dst, zeros_hid, n_pad, k, d_hid)
  h2 = _tc_mid(agg, deg, b3r, w4p, n)
  agg = _sc_segment_sum(h2, src, dst, zeros_32, n_pad, k, 32)
  return _tc_final(agg, deg, b4r, n, d_out)


# speedup vs baseline: 2.3215x; 2.3215x over previous
import functools

import jax
import jax.numpy as jnp
from jax import lax
from jax.experimental import pallas as pl
from jax.experimental.pallas import tpu as pltpu
from jax.experimental.pallas import tpu_sc as plsc

NC = 2
NS = 16
NW = NC * NS
CHUNK = 128


def _sc_segment_sum(h2, src_idx, dst_idx, zeros, n_pad, k, d):
  rpz = n_pad // NS
  mesh = plsc.VectorSubcoreMesh(core_axis_name="c", subcore_axis_name="s")

  @functools.partial(
      pl.kernel,
      out_type=jax.ShapeDtypeStruct((NC, n_pad, d), jnp.float32),
      mesh=mesh,
      scratch_types=[
          pltpu.VMEM((k, CHUNK), jnp.int32),
          pltpu.VMEM((k, CHUNK), jnp.int32),
          pltpu.VMEM((CHUNK, d), jnp.float32),
          pltpu.VMEM_SHARED((n_pad, d), jnp.float32),
      ],
  )
  def body(h2_hbm, src_hbm, dst_hbm, zero_hbm, out_hbm, src_v, dst_v, rows_v,
           agg_sh):
    c = lax.axis_index("c")
    s = lax.axis_index("s")
    wid = c * NS + s
    pltpu.sync_copy(src_hbm.at[wid], src_v)
    pltpu.sync_copy(dst_hbm.at[wid], dst_v)
    pltpu.sync_copy(zero_hbm, agg_sh.at[pl.ds(s * rpz, rpz)])
    plsc.subcore_barrier()

    @pl.loop(0, k)
    def _(j):
      pltpu.sync_copy(h2_hbm.at[src_v.at[j]], rows_v)
      pltpu.sync_copy(rows_v, agg_sh.at[dst_v.at[j]], add=True)

    plsc.subcore_barrier()
    pltpu.sync_copy(agg_sh.at[pl.ds(s * rpz, rpz)],
                    out_hbm.at[c, pl.ds(s * rpz, rpz)])

  return body(h2, src_idx, dst_idx, zeros)


def _sc_degree(dst_idx, ones, zeros, n_pad, k):
  rpz = n_pad // NS
  mesh = plsc.VectorSubcoreMesh(core_axis_name="c", subcore_axis_name="s")

  @functools.partial(
      pl.kernel,
      out_type=jax.ShapeDtypeStruct((NC, n_pad, 16), jnp.float32),
      mesh=mesh,
      scratch_types=[
          pltpu.VMEM((k, CHUNK), jnp.int32),
          pltpu.VMEM((CHUNK, 16), jnp.float32),
          pltpu.VMEM_SHARED((n_pad, 16), jnp.float32),
      ],
  )
  def body(dst_hbm, ones_hbm, zero_hbm, out_hbm, dst_v, ones_v, deg_sh):
    c = lax.axis_index("c")
    s = lax.axis_index("s")
    wid = c * NS + s
    pltpu.sync_copy(dst_hbm.at[wid], dst_v)
    pltpu.sync_copy(ones_hbm, ones_v)
    pltpu.sync_copy(zero_hbm, deg_sh.at[pl.ds(s * rpz, rpz)])
    plsc.subcore_barrier()

    @pl.loop(0, k)
    def _(j):
      pltpu.sync_copy(ones_v, deg_sh.at[dst_v.at[j]], add=True)

    plsc.subcore_barrier()
    pltpu.sync_copy(deg_sh.at[pl.ds(s * rpz, rpz)],
                    out_hbm.at[c, pl.ds(s * rpz, rpz)])

  return body(dst_idx, ones, zeros)


def _tc_matmul(x, w):
  def body(x_ref, w_ref, o_ref):
    o_ref[...] = jnp.dot(x_ref[...], w_ref[...],
                         preferred_element_type=jnp.float32)

  return pl.pallas_call(
      body,
      out_shape=jax.ShapeDtypeStruct((x.shape[0], w.shape[1]), jnp.float32),
  )(x, w)


def _tc_scale(p, deg, n):
  def body(p_ref, deg_ref, o_ref):
    dinv = lax.rsqrt(deg_ref[0, :n, 0:1] + deg_ref[1, :n, 0:1])
    o_ref[...] = dinv * p_ref[...]

  return pl.pallas_call(
      body,
      out_shape=jax.ShapeDtypeStruct(p.shape, jnp.float32),
  )(p, deg)


def _tc_mid(agg, deg, b, w, n):
  def body(agg_ref, deg_ref, b_ref, w_ref, o_ref):
    dinv = lax.rsqrt(deg_ref[0, :n, 0:1] + deg_ref[1, :n, 0:1])
    h = jnp.maximum(dinv * (agg_ref[0, :n, :] + agg_ref[1, :n, :])
                    + b_ref[...], 0.0)
    o_ref[...] = dinv * jnp.dot(h, w_ref[...],
                                preferred_element_type=jnp.float32)

  return pl.pallas_call(
      body,
      out_shape=jax.ShapeDtypeStruct((n, w.shape[1]), jnp.float32),
  )(agg, deg, b, w)


def _tc_final(agg, deg, b, n, d_out):
  def body(agg_ref, deg_ref, b_ref, o_ref):
    dinv = lax.rsqrt(deg_ref[0, :n, 0:1] + deg_ref[1, :n, 0:1])
    o_ref[...] = jax.nn.sigmoid(
        dinv * (agg_ref[0, :n, :d_out] + agg_ref[1, :n, :d_out]) + b_ref[...])

  return pl.pallas_call(
      body,
      out_shape=jax.ShapeDtypeStruct((n, d_out), jnp.float32),
  )(agg, deg, b)


def kernel(x, edge_index, W1, b1, W2, b2, W3, b3, W4, b4):
  n, d_in = x.shape
  e = edge_index.shape[1]
  d_hid = W1.shape[1]
  d_out = W4.shape[1]

  tot = e + n
  per = NW * CHUNK
  k = -(-tot // per)
  k = -(-k // 8) * 8
  e_pad = k * per
  n_pad = -(-n // (NS * CHUNK)) * (NS * CHUNK)

  loop = jnp.arange(n, dtype=jnp.int32)
  src = jnp.concatenate([edge_index[0].astype(jnp.int32), loop])
  dst = jnp.concatenate([edge_index[1].astype(jnp.int32), loop])
  npad_e = e_pad - tot
  pad_dst = n + (jnp.arange(npad_e, dtype=jnp.int32) % (n_pad - n))
  src = jnp.pad(src, (0, npad_e)).reshape(NW, k, CHUNK)
  dst = jnp.concatenate([dst, pad_dst]).reshape(NW, k, CHUNK)

  rpz = n_pad // NS
  zeros_hid = jnp.zeros((rpz, d_hid), jnp.float32)
  zeros_16 = jnp.zeros((rpz, 16), jnp.float32)
  ones_16 = jnp.ones((CHUNK, 16), jnp.float32)

  b1r = b1.reshape(1, -1)
  b2r = b2.reshape(1, -1)
  b3r = b3.reshape(1, -1)
  b4r = b4.reshape(1, -1)
  w4p = jnp.pad(W4, ((0, 0), (0, 128 - d_out)))

  deg = _sc_degree(dst, ones_16, zeros_16, n_pad, k)
  p1 = _tc_matmul(x, W1)
  h2 = _tc_scale(p1, deg, n)
  agg = _sc_segment_sum(h2, src, dst, zeros_hid, n_pad, k, d_hid)
  h2 = _tc_mid(agg, deg, b1r, W2, n)
  agg = _sc_segment_sum(h2, src, dst, zeros_hid, n_pad, k, d_hid)
  h2 = _tc_mid(agg, deg, b2r, W3, n)
  agg = _sc_segment_sum(h2, src, dst, zeros_hid, n_pad, k, d_hid)
  h2 = _tc_mid(agg, deg, b3r, w4p, n)
  agg = _sc_segment_sum(h2, src, dst, zeros_hid, n_pad, k, 128)
  return _tc_final(agg, deg, b4r, n, d_out)
